# baseline (speedup 1.0000x reference)
import functools

import jax
import jax.numpy as jnp
from jax import lax
from jax.experimental import pallas as pl
from jax.experimental.pallas import tpu as pltpu
from jax.experimental.pallas import tpu_sc as plsc

UT = 130305
NROWS = 96
ROWS_PER_TILE = 3
GSZ = 65152
IB = 8192
BUF = 8448
NIB = 16
LASTIB = UT - 1 - (15 * IB - 128)
M_INIT = 363


def _make_sc_call():
  mesh = plsc.VectorSubcoreMesh(core_axis_name="c", subcore_axis_name="s")

  @functools.partial(
      pl.kernel,
      out_type=jax.ShapeDtypeStruct((NROWS, UT), jnp.float32),
      mesh=mesh,
      compiler_params=pltpu.CompilerParams(needs_layout_passes=False),
      scratch_types=[
          pltpu.VMEM((GSZ,), jnp.float32),
          pltpu.VMEM((2 * BUF,), jnp.float32),
          pltpu.SemaphoreType.DMA,
          pltpu.SemaphoreType.DMA,
          pltpu.SemaphoreType.DMA,
      ],
  )
  def sc_permute(x_hbm, out_hbm, obuf, ibuf, isem0, isem1, hsem):
    cid = lax.axis_index("c")
    sid = lax.axis_index("s")
    wid = sid * 2 + cid
    lane = lax.iota(jnp.int32, 16)

    def do_task(q, _):
      row = wid * ROWS_PER_TILE + (q >> 1)
      group = q & 1
      ka = group * GSZ

      def sync_in(n):
        def cp(src_off, ln):
          for p in range(2):
            @pl.when((n & 1) == p)
            def _(p=p):
              pltpu.sync_copy(
                  x_hbm.at[row, pl.ds(src_off, ln)],
                  ibuf.at[pl.ds(p * BUF, ln)])

        @pl.when(n == 0)
        def _():
          cp(0, BUF)

        @pl.when((n >= 1) & (n <= 14))
        def _():
          cp(n * IB - 128, BUF)

        @pl.when(n == 15)
        def _():
          cp(15 * IB - 128, LASTIB)

      @pl.when(q >= 1)
      def _():
        pltpu.make_async_copy(
            obuf, out_hbm.at[row, pl.ds(0, GSZ)], hsem).wait()

      sync_in(0)

      def out_row(r, carry):
        m_bound, st, n = carry
        m_bound = lax.while_loop(
            lambda m: (m > 2)
            & ((((511 - (m - 1)) * (510 + (m - 1))) >> 1) + m - 3
               < GSZ + r),
            lambda m: m - 1, m_bound)

        w_lo = jnp.where(group == 0,
                         jnp.maximum(m_bound, r + 2),
                         jnp.maximum(r + 2, 3))
        w_hi = jnp.where(group == 0, 512, m_bound)
        width = w_hi - w_lo
        f0 = st + w_lo - r - 2
        cv = w_lo + lane
        dest0 = ((511 - cv) * (510 + cv) >> 1) + cv - 2 - r - ka
        dd0 = -104 - 16 * cv
        nfull = (width - 1) >> 4

        def chunk(_, cc):
          dest, dd, f, nn = cc
          sw = f > nn * IB + 8304
          nn = jnp.where(sw, nn + 1, nn)

          @pl.when(sw)
          def _():
            sync_in(nn)

          base = jnp.where(nn == 0, 0, nn * IB - 128)
          addr = (nn & 1) * BUF + (f - base)
          vals = plsc.load_gather(ibuf, [addr + lane])
          plsc.store_scatter(obuf, [dest], vals)
          return (dest + dd, dd - 256, f + 16, nn)

        dest, dd, f, n = lax.fori_loop(
            0, nfull, chunk, (dest0, dd0, f0, n))

        f_l = f0 + width - 16
        sw = f_l > n * IB + 8304
        n = jnp.where(sw, n + 1, n)

        @pl.when(sw)
        def _():
          sync_in(n)

        cl = (w_hi - 16) + lane
        dest_l = ((511 - cl) * (510 + cl) >> 1) + cl - 2 - r - ka
        base = jnp.where(n == 0, 0, n * IB - 128)
        addr = (n & 1) * BUF + jnp.clip(f_l - base, 0, BUF - 16)
        vals = plsc.load_gather(ibuf, [addr + lane])
        plsc.store_scatter(obuf, [dest_l], vals,
                           mask=lane >= 16 - width)
        return (m_bound, st + 510 - r, n)

      _, _, n = lax.fori_loop(0, 510, out_row, (M_INIT, 0, 0))

      pltpu.async_copy(obuf, out_hbm.at[row, pl.ds(ka, GSZ)], hsem)
      return 0

    lax.fori_loop(0, 2 * ROWS_PER_TILE, do_task, 0)
    pltpu.make_async_copy(
        obuf, out_hbm.at[0, pl.ds(0, GSZ)], hsem).wait()

  return sc_permute


_SC_PERMUTE = _make_sc_call()


def kernel(x, reverse):
  bs, num, ut = x.shape

  def do_reverse(xx):
    flat = xx.reshape(NROWS, UT)
    out = _SC_PERMUTE(flat)
    out = out.at[:, UT - 1].set(flat[:, 0])
    out = out.at[:, 0].set(flat[:, UT - 1])
    return out.reshape(bs, num, ut)

  return lax.cond(jnp.asarray(reverse) != 0, do_reverse, lambda xx: xx, x)


# speedup vs baseline: 1.1709x; 1.1709x over previous
import functools

import jax
import jax.numpy as jnp
from jax import lax
from jax.experimental import pallas as pl
from jax.experimental.pallas import tpu as pltpu
from jax.experimental.pallas import tpu_sc as plsc

UT = 130305
NROWS = 96
ROWS_PER_TILE = 3
GSZ = 65152
IB = 8192
BUF = 8448
NIB = 16
LASTIB = UT - 1 - (15 * IB - 128)
M_INIT = 363


def _make_sc_call():
  mesh = plsc.VectorSubcoreMesh(core_axis_name="c", subcore_axis_name="s")

  @functools.partial(
      pl.kernel,
      out_type=jax.ShapeDtypeStruct((NROWS, UT), jnp.float32),
      mesh=mesh,
      compiler_params=pltpu.CompilerParams(needs_layout_passes=False),
      scratch_types=[
          pltpu.VMEM((GSZ,), jnp.float32),
          pltpu.VMEM((2 * BUF,), jnp.float32),
          pltpu.SemaphoreType.DMA,
          pltpu.SemaphoreType.DMA,
          pltpu.SemaphoreType.DMA,
      ],
  )
  def sc_permute(x_hbm, out_hbm, obuf, ibuf, isem0, isem1, hsem):
    cid = lax.axis_index("c")
    sid = lax.axis_index("s")
    wid = sid * 2 + cid
    lane = lax.iota(jnp.int32, 16)

    def do_task(q, _):
      row = wid * ROWS_PER_TILE + (q >> 1)
      group = q & 1
      ka = group * GSZ

      def fire_in(n):
        def cp(src_off, ln):
          for p in range(2):
            @pl.when((n & 1) == p)
            def _(p=p):
              pltpu.async_copy(
                  x_hbm.at[row, pl.ds(src_off, ln)],
                  ibuf.at[pl.ds(p * BUF, ln)],
                  isem1 if p else isem0)

        @pl.when(n == 0)
        def _():
          cp(0, BUF)

        @pl.when((n >= 1) & (n <= 14))
        def _():
          cp(n * IB - 128, BUF)

        @pl.when(n == 15)
        def _():
          cp(15 * IB - 128, LASTIB)

      def wait_in(n):
        def wt(ln):
          for p in range(2):
            @pl.when((n & 1) == p)
            def _(p=p):
              pltpu.make_async_copy(
                  x_hbm.at[row, pl.ds(0, ln)],
                  ibuf.at[pl.ds(p * BUF, ln)],
                  isem1 if p else isem0).wait()

        @pl.when(n <= 14)
        def _():
          wt(BUF)

        @pl.when(n == 15)
        def _():
          wt(LASTIB)

      def enter_block(n):
        wait_in(n)

        @pl.when(n <= 14)
        def _():
          fire_in(n + 1)

      fire_in(0)
      fire_in(1)

      @pl.when(q >= 1)
      def _():
        pltpu.make_async_copy(
            obuf, out_hbm.at[row, pl.ds(0, GSZ)], hsem).wait()

      wait_in(0)

      def out_row(r, carry):
        m_bound, st, n = carry
        m_bound = lax.while_loop(
            lambda m: (m > 2)
            & ((((511 - (m - 1)) * (510 + (m - 1))) >> 1) + m - 3
               < GSZ + r),
            lambda m: m - 1, m_bound)

        w_lo = jnp.where(group == 0,
                         jnp.maximum(m_bound, r + 2),
                         jnp.maximum(r + 2, 3))
        w_hi = jnp.where(group == 0, 512, m_bound)
        width = w_hi - w_lo
        f0 = st + w_lo - r - 2
        cv = w_lo + lane
        dest0 = ((511 - cv) * (510 + cv) >> 1) + cv - 2 - r - ka
        dd0 = -104 - 16 * cv
        nfull = (width - 1) >> 4

        def chunk(_, cc):
          dest, dd, f, nn = cc
          sw = f > nn * IB + 8304
          nn = jnp.where(sw, nn + 1, nn)

          @pl.when(sw)
          def _():
            enter_block(nn)

          base = jnp.where(nn == 0, 0, nn * IB - 128)
          addr = (nn & 1) * BUF + (f - base)
          vals = plsc.load_gather(ibuf, [addr + lane])
          plsc.store_scatter(obuf, [dest], vals)
          return (dest + dd, dd - 256, f + 16, nn)

        dest, dd, f, n = lax.fori_loop(
            0, nfull, chunk, (dest0, dd0, f0, n))

        f_l = f0 + width - 16
        sw = f_l > n * IB + 8304
        n = jnp.where(sw, n + 1, n)

        @pl.when(sw)
        def _():
          enter_block(n)

        cl = (w_hi - 16) + lane
        dest_l = ((511 - cl) * (510 + cl) >> 1) + cl - 2 - r - ka
        base = jnp.where(n == 0, 0, n * IB - 128)
        addr = (n & 1) * BUF + jnp.clip(f_l - base, 0, BUF - 16)
        vals = plsc.load_gather(ibuf, [addr + lane])
        plsc.store_scatter(obuf, [dest_l], vals,
                           mask=lane >= 16 - width)
        return (m_bound, st + 510 - r, n)

      _, _, n = lax.fori_loop(0, 510, out_row, (M_INIT, 0, 0))

      @pl.when(n < 15)
      def _():
        wait_in(n + 1)

      pltpu.async_copy(obuf, out_hbm.at[row, pl.ds(ka, GSZ)], hsem)
      return 0

    lax.fori_loop(0, 2 * ROWS_PER_TILE, do_task, 0)
    pltpu.make_async_copy(
        obuf, out_hbm.at[0, pl.ds(0, GSZ)], hsem).wait()

  return sc_permute


_SC_PERMUTE = _make_sc_call()


def kernel(x, reverse):
  bs, num, ut = x.shape

  def do_reverse(xx):
    flat = xx.reshape(NROWS, UT)
    out = _SC_PERMUTE(flat)
    out = out.at[:, UT - 1].set(flat[:, 0])
    out = out.at[:, 0].set(flat[:, UT - 1])
    return out.reshape(bs, num, ut)

  return lax.cond(jnp.asarray(reverse) != 0, do_reverse, lambda xx: xx, x)
